# baseline (speedup 1.0000x reference)
import jax
import jax.numpy as jnp
from jax import lax
from jax.experimental import pallas as pl
from jax.experimental.pallas import tpu as pltpu
from jax.experimental.pallas import tpu_sc as plsc

NUM_USERS = 1000
NUM_ITEMS = 1000
RANK = 128
RATING_MEAN = 3.5
BATCH = 16384

NC = 2
NS = 16
L = 16
NW = NC * NS
PW = BATCH // NW
C = 128
NCHUNK = PW // C


def _mf_body(eu, ei, um, im, uid2, iid2, out,
             us_sh, is_sh, uidx2, iidx2,
             ur0, ir0, ur1, ir1, umv, imv, outv,
             sem0, sem0b, sem1, sem1b):
    c = lax.axis_index("c")
    s = lax.axis_index("s")
    wid = s * NC + c
    base = wid * PW

    @pl.when(s == 0)
    def _stage_i():
        pltpu.sync_copy(ei, is_sh)

    @pl.when(s == 1)
    def _stage_u():
        pltpu.sync_copy(eu, us_sh)

    pltpu.sync_copy(um, umv)
    pltpu.sync_copy(im, imv)
    pltpu.sync_copy(uid2.at[pl.ds(wid * NCHUNK, NCHUNK)], uidx2)
    pltpu.sync_copy(iid2.at[pl.ds(wid * NCHUNK, NCHUNK)], iidx2)

    for ch in range(NCHUNK):
        for j in range(C // L):
            sl = pl.ds(j * L, L)
            uidx2[ch, sl] = uidx2[ch, sl] + 1
            iidx2[ch, sl] = iidx2[ch, sl] + 1

    riota = lax.broadcasted_iota(jnp.int32, (L,), 0)
    zero = jnp.zeros((L,), jnp.float32)
    bufs = [(ur0, ir0, sem0, sem0b), (ur1, ir1, sem1, sem1b)]

    def fire_u(ch, from_hbm):
        ur, _, sem, _ = bufs[ch % 2]
        src_tab = eu if from_hbm else us_sh
        return pltpu.async_copy(src_tab.at[uidx2.at[ch]], ur, sem)

    def fire_i(ch):
        _, ir, _, semb = bufs[ch % 2]
        return pltpu.async_copy(is_sh.at[iidx2.at[ch]], ir, semb)

    cu0 = fire_u(0, True)
    cu1 = fire_u(1, True)
    plsc.subcore_barrier()
    pend = (cu0, fire_i(0))
    nxt_u = cu1
    for ch in range(NCHUNK):
        nxt = (nxt_u, fire_i(ch + 1)) if ch + 1 < NCHUNK else None
        pend[0].wait()
        pend[1].wait()
        urows, irows = bufs[ch % 2][0], bufs[ch % 2][1]

        def group_body(g, _, urows=urows, irows=irows, ch=ch):
            rows = riota + g * L

            def dot_step(t, carry):
                col, a0, a1, a2, a3 = carry
                c1 = (col + 1) & (RANK - 1)
                c2 = (col + 2) & (RANK - 1)
                c3 = (col + 3) & (RANK - 1)
                u0 = plsc.load_gather(urows, [rows, col])
                i0 = plsc.load_gather(irows, [rows, col])
                u1 = plsc.load_gather(urows, [rows, c1])
                i1 = plsc.load_gather(irows, [rows, c1])
                u2 = plsc.load_gather(urows, [rows, c2])
                i2 = plsc.load_gather(irows, [rows, c2])
                u3 = plsc.load_gather(urows, [rows, c3])
                i3 = plsc.load_gather(irows, [rows, c3])
                return ((col + 4) & (RANK - 1), a0 + u0 * i0, a1 + u1 * i1,
                        a2 + u2 * i2, a3 + u3 * i3)

            _, a0, a1, a2, a3 = lax.fori_loop(
                0, RANK // 4, dot_step,
                (riota, zero, zero, zero, zero))
            dot = (a0 + a1) + (a2 + a3)

            goff = pl.multiple_of(g * L, L)
            umean = plsc.load_gather(umv, [uidx2[ch, pl.ds(goff, L)]])
            imean = plsc.load_gather(imv, [iidx2[ch, pl.ds(goff, L)]])
            outv[pl.ds(ch * C + goff, L)] = dot + umean + imean + RATING_MEAN
            return 0

        lax.fori_loop(0, C // L, group_body, 0)
        nxt_u = fire_u(ch + 2, ch + 2 < 3) if ch + 2 < NCHUNK else None
        pend = nxt

    pltpu.sync_copy(outv, out.at[pl.ds(base, PW)])


@jax.jit
def _mf_sc(eu, ei, um, im, uid2, iid2):
    mesh = plsc.VectorSubcoreMesh(
        core_axis_name="c", subcore_axis_name="s",
        num_cores=NC, num_subcores=NS)
    run = pl.kernel(
        _mf_body,
        out_type=jax.ShapeDtypeStruct((BATCH,), jnp.float32),
        mesh=mesh,
        scratch_types=[
            pltpu.VMEM_SHARED((NUM_USERS + 1, RANK), jnp.float32),
            pltpu.VMEM_SHARED((NUM_ITEMS + 1, RANK), jnp.float32),
            pltpu.VMEM((NCHUNK, C), jnp.int32),
            pltpu.VMEM((NCHUNK, C), jnp.int32),
            pltpu.VMEM((C, RANK), jnp.float32),
            pltpu.VMEM((C, RANK), jnp.float32),
            pltpu.VMEM((C, RANK), jnp.float32),
            pltpu.VMEM((C, RANK), jnp.float32),
            pltpu.VMEM((NUM_USERS + 1,), jnp.float32),
            pltpu.VMEM((NUM_ITEMS + 1,), jnp.float32),
            pltpu.VMEM((PW,), jnp.float32),
            pltpu.SemaphoreType.DMA,
            pltpu.SemaphoreType.DMA,
            pltpu.SemaphoreType.DMA,
            pltpu.SemaphoreType.DMA,
        ],
        compiler_params=pltpu.CompilerParams(needs_layout_passes=False),
    )
    return run(eu, ei, um, im, uid2, iid2)


def kernel(emb_user, emb_item, emb_user_mean, emb_item_mean,
           user_id, item_id, user_vocab, item_vocab):
    um = emb_user_mean.reshape(-1)
    im = emb_item_mean.reshape(-1)
    out = _mf_sc(emb_user, emb_item, um, im,
                 user_id.reshape(NW * NCHUNK, C),
                 item_id.reshape(NW * NCHUNK, C))
    return out.reshape(BATCH, 1, 1)


# speedup vs baseline: 1.0216x; 1.0216x over previous
import jax
import jax.numpy as jnp
from jax import lax
from jax.experimental import pallas as pl
from jax.experimental.pallas import tpu as pltpu
from jax.experimental.pallas import tpu_sc as plsc

NUM_USERS = 1000
NUM_ITEMS = 1000
RANK = 128
RATING_MEAN = 3.5
BATCH = 16384

NC = 2
NS = 16
L = 16
NW = NC * NS
PW = BATCH // NW
C = 128
NCHUNK = PW // C


def _mf_body(eu, ei, um, im, uid2, iid2, out,
             is_sh, uidx2, iidx2,
             ur0, ir0, ur1, ir1, umv, imv, outv,
             sem0, sem0b, sem1, sem1b):
    c = lax.axis_index("c")
    s = lax.axis_index("s")
    wid = s * NC + c
    base = wid * PW

    @pl.when(s == 0)
    def _stage():
        pltpu.sync_copy(ei, is_sh)

    pltpu.sync_copy(um, umv)
    pltpu.sync_copy(im, imv)
    pltpu.sync_copy(uid2.at[pl.ds(wid * NCHUNK, NCHUNK)], uidx2)
    pltpu.sync_copy(iid2.at[pl.ds(wid * NCHUNK, NCHUNK)], iidx2)

    for ch in range(NCHUNK):
        for j in range(C // L):
            sl = pl.ds(j * L, L)
            uidx2[ch, sl] = uidx2[ch, sl] + 1
            iidx2[ch, sl] = iidx2[ch, sl] + 1

    riota = lax.broadcasted_iota(jnp.int32, (L,), 0)
    zero = jnp.zeros((L,), jnp.float32)
    bufs = [(ur0, ir0, sem0, sem0b), (ur1, ir1, sem1, sem1b)]

    def fire_u(ch):
        ur, _, sem, _ = bufs[ch % 2]
        return pltpu.async_copy(eu.at[uidx2.at[ch]], ur, sem)

    def fire_i(ch):
        _, ir, _, semb = bufs[ch % 2]
        return pltpu.async_copy(is_sh.at[iidx2.at[ch]], ir, semb)

    cu0 = fire_u(0)
    cu1 = fire_u(1)
    plsc.subcore_barrier()
    pend = (cu0, fire_i(0))
    nxt_u = cu1
    for ch in range(NCHUNK):
        nxt = (nxt_u, fire_i(ch + 1)) if ch + 1 < NCHUNK else None
        pend[0].wait()
        pend[1].wait()
        urows, irows = bufs[ch % 2][0], bufs[ch % 2][1]

        def group_body(g, _, urows=urows, irows=irows, ch=ch):
            rows = riota + g * L

            def dot_step(t, carry):
                col, a0, a1, a2, a3 = carry
                c1 = (col + 1) & (RANK - 1)
                c2 = (col + 2) & (RANK - 1)
                c3 = (col + 3) & (RANK - 1)
                u0 = plsc.load_gather(urows, [rows, col])
                i0 = plsc.load_gather(irows, [rows, col])
                u1 = plsc.load_gather(urows, [rows, c1])
                i1 = plsc.load_gather(irows, [rows, c1])
                u2 = plsc.load_gather(urows, [rows, c2])
                i2 = plsc.load_gather(irows, [rows, c2])
                u3 = plsc.load_gather(urows, [rows, c3])
                i3 = plsc.load_gather(irows, [rows, c3])
                return ((col + 4) & (RANK - 1), a0 + u0 * i0, a1 + u1 * i1,
                        a2 + u2 * i2, a3 + u3 * i3)

            _, a0, a1, a2, a3 = lax.fori_loop(
                0, RANK // 4, dot_step,
                (riota, zero, zero, zero, zero))
            dot = (a0 + a1) + (a2 + a3)

            goff = pl.multiple_of(g * L, L)
            umean = plsc.load_gather(umv, [uidx2[ch, pl.ds(goff, L)]])
            imean = plsc.load_gather(imv, [iidx2[ch, pl.ds(goff, L)]])
            outv[pl.ds(ch * C + goff, L)] = dot + umean + imean + RATING_MEAN
            return 0

        lax.fori_loop(0, C // L, group_body, 0)
        nxt_u = fire_u(ch + 2) if ch + 2 < NCHUNK else None
        pend = nxt

    pltpu.sync_copy(outv, out.at[pl.ds(base, PW)])


@jax.jit
def _mf_sc(eu, ei, um, im, uid2, iid2):
    mesh = plsc.VectorSubcoreMesh(
        core_axis_name="c", subcore_axis_name="s",
        num_cores=NC, num_subcores=NS)
    run = pl.kernel(
        _mf_body,
        out_type=jax.ShapeDtypeStruct((BATCH,), jnp.float32),
        mesh=mesh,
        scratch_types=[
            pltpu.VMEM_SHARED((NUM_ITEMS + 1, RANK), jnp.float32),
            pltpu.VMEM((NCHUNK, C), jnp.int32),
            pltpu.VMEM((NCHUNK, C), jnp.int32),
            pltpu.VMEM((C, RANK), jnp.float32),
            pltpu.VMEM((C, RANK), jnp.float32),
            pltpu.VMEM((C, RANK), jnp.float32),
            pltpu.VMEM((C, RANK), jnp.float32),
            pltpu.VMEM((NUM_USERS + 1,), jnp.float32),
            pltpu.VMEM((NUM_ITEMS + 1,), jnp.float32),
            pltpu.VMEM((PW,), jnp.float32),
            pltpu.SemaphoreType.DMA,
            pltpu.SemaphoreType.DMA,
            pltpu.SemaphoreType.DMA,
            pltpu.SemaphoreType.DMA,
        ],
        compiler_params=pltpu.CompilerParams(needs_layout_passes=False),
    )
    return run(eu, ei, um, im, uid2, iid2)


def kernel(emb_user, emb_item, emb_user_mean, emb_item_mean,
           user_id, item_id, user_vocab, item_vocab):
    um = emb_user_mean.reshape(-1)
    im = emb_item_mean.reshape(-1)
    out = _mf_sc(emb_user, emb_item, um, im,
                 user_id.reshape(NW * NCHUNK, C),
                 item_id.reshape(NW * NCHUNK, C))
    return out.reshape(BATCH, 1, 1)
